# Initial kernel scaffold; baseline (speedup 1.0000x reference)
import jax
import jax.numpy as jnp
from jax.experimental import pallas as pl


def kernel(node, edge, edge_index, Wn, bn, We, be, Wa, ba, Wt, bt):
    raise NotImplementedError("write your pallas kernel here")


# speedup vs baseline: 2.3785x; 2.3785x over previous
import functools

import jax
import jax.numpy as jnp
from jax import lax
from jax.experimental import pallas as pl
from jax.experimental.pallas import tpu as pltpu
from jax.experimental.pallas import tpu_sc as plsc

_NC = 2
_NS = 16
_CH = 128


def _node_precompute(node, Wn, bn2, We_top):
    n, d = node.shape
    blk = 1000
    assert n % blk == 0

    def body(node_ref, wn_ref, bn_ref, wet_ref, out_ref):
        xb = node_ref[...].astype(jnp.bfloat16)
        wn = wn_ref[...].astype(jnp.bfloat16)
        h = jnp.dot(xb, wn, preferred_element_type=jnp.float32) + bn_ref[...]
        h = jnp.where(h >= 0, h, 0.01 * h)
        wet = wet_ref[...].astype(jnp.bfloat16)
        out_ref[...] = jnp.dot(h.astype(jnp.bfloat16), wet,
                               preferred_element_type=jnp.float32)

    return pl.pallas_call(
        body,
        grid=(n // blk,),
        in_specs=[
            pl.BlockSpec((blk, d), lambda i: (i, 0)),
            pl.BlockSpec((d, d), lambda i: (0, 0)),
            pl.BlockSpec((1, d), lambda i: (0, 0)),
            pl.BlockSpec((d, d), lambda i: (0, 0)),
        ],
        out_specs=pl.BlockSpec((blk, d), lambda i: (i, 0)),
        out_shape=jax.ShapeDtypeStruct((n, d), jnp.float32),
    )(node, Wn, bn2, We_top)


def _gather_rows(table, idx):
    n, d = table.shape
    (e,) = idx.shape
    n_chunks = e // _CH
    assert n_chunks * _CH == e
    nw = _NC * _NS
    per_w = (n_chunks + nw - 1) // nw
    mesh = plsc.VectorSubcoreMesh(core_axis_name="c", subcore_axis_name="s")

    @functools.partial(
        pl.kernel,
        out_type=jax.ShapeDtypeStruct((e, d), jnp.float32),
        mesh=mesh,
        scratch_types=[
            pltpu.VMEM((_CH,), jnp.int32),
            pltpu.VMEM((_CH, d), jnp.float32),
            pltpu.SemaphoreType.DMA,
        ],
    )
    def k(table_hbm, idx_hbm, out_hbm, idx_v, rows_v, sem):
        wid = lax.axis_index("s") * _NC + lax.axis_index("c")

        def body(i, carry):
            c = i * nw + wid

            @pl.when(c < n_chunks)
            def _():
                base = c * _CH
                pltpu.sync_copy(idx_hbm.at[pl.ds(base, _CH)], idx_v)
                pltpu.async_copy(table_hbm.at[idx_v], rows_v, sem).wait()
                pltpu.sync_copy(rows_v, out_hbm.at[pl.ds(base, _CH)])

            return carry

        lax.fori_loop(0, per_w, body, 0)

    return k(table, idx)


def _edge_compute(G, edge, We_bot, be2, Wt, bt2):
    e, d = edge.shape
    blk = 1280
    assert e % blk == 0

    def body(g_ref, e_ref, web_ref, be_ref, wt_ref, bt_ref, out_ref):
        eb = e_ref[...].astype(jnp.bfloat16)
        web = web_ref[...].astype(jnp.bfloat16)
        m = jnp.dot(eb, web, preferred_element_type=jnp.float32)
        m = m + g_ref[...] + be_ref[...]
        m = jnp.where(m >= 0, m, 0.01 * m)
        wt = wt_ref[...].astype(jnp.bfloat16)
        out_ref[...] = jnp.dot(m.astype(jnp.bfloat16), wt,
                               preferred_element_type=jnp.float32) + bt_ref[...]

    return pl.pallas_call(
        body,
        grid=(e // blk,),
        in_specs=[
            pl.BlockSpec((blk, d), lambda i: (i, 0)),
            pl.BlockSpec((blk, d), lambda i: (i, 0)),
            pl.BlockSpec((d, d), lambda i: (0, 0)),
            pl.BlockSpec((1, d), lambda i: (0, 0)),
            pl.BlockSpec((d, d), lambda i: (0, 0)),
            pl.BlockSpec((1, d), lambda i: (0, 0)),
        ],
        out_specs=pl.BlockSpec((blk, d), lambda i: (i, 0)),
        out_shape=jax.ShapeDtypeStruct((e, d), jnp.float32),
    )(G, edge, We_bot, be2, Wt, bt2)


def _scatter_add_relu(ac, dst, n):
    e, d = ac.shape
    dh = d // _NC
    n_chunks = e // _CH
    assert n_chunks * _CH == e
    per_t = (n_chunks + _NS - 1) // _NS
    wb = 80
    n_blk = n // wb
    assert n_blk * wb == n
    per_wb = (n_blk + _NS - 1) // _NS
    mesh = plsc.VectorSubcoreMesh(core_axis_name="c", subcore_axis_name="s")

    @functools.partial(
        pl.kernel,
        out_type=jax.ShapeDtypeStruct((n, d), jnp.float32),
        mesh=mesh,
        scratch_types=[
            pltpu.VMEM((_CH,), jnp.int32),
            pltpu.VMEM((_CH, dh), jnp.float32),
            pltpu.VMEM((wb, dh), jnp.float32),
            pltpu.VMEM_SHARED((n, dh), jnp.float32),
            pltpu.SemaphoreType.DMA,
        ],
    )
    def k(ac_hbm, dst_hbm, out_hbm, idx_v, rows_v, buf_v, acc_sh, sem):
        cid = lax.axis_index("c")
        sid = lax.axis_index("s")

        def zbody(i, carry):
            for j in range(dh // 16):
                buf_v[i, pl.ds(j * 16, 16)] = jnp.zeros((16,), jnp.float32)
            return carry

        lax.fori_loop(0, wb, zbody, 0)

        def zcopy(i, carry):
            g = i * _NS + sid

            @pl.when(g < n_blk)
            def _():
                pltpu.sync_copy(buf_v, acc_sh.at[pl.ds(g * wb, wb)])

            return carry

        lax.fori_loop(0, per_wb, zcopy, 0)
        plsc.subcore_barrier()

        def body(i, carry):
            c = i * _NS + sid

            @pl.when(c < n_chunks)
            def _():
                base = c * _CH
                pltpu.sync_copy(dst_hbm.at[pl.ds(base, _CH)], idx_v)
                pltpu.sync_copy(
                    ac_hbm.at[pl.ds(base, _CH), pl.ds(cid * dh, dh)], rows_v)
                pltpu.sync_copy(rows_v, acc_sh.at[idx_v], add=True)

            return carry

        lax.fori_loop(0, per_t, body, 0)
        plsc.subcore_barrier()

        def wcopy(i, carry):
            g = i * _NS + sid

            @pl.when(g < n_blk)
            def _():
                r0 = g * wb
                pltpu.sync_copy(acc_sh.at[pl.ds(r0, wb)], buf_v)

                def rbody(ii, cc):
                    for j in range(dh // 16):
                        sl = pl.ds(j * 16, 16)
                        buf_v[ii, sl] = jnp.maximum(buf_v[ii, sl], 0.0)
                    return cc

                lax.fori_loop(0, wb, rbody, 0)
                pltpu.sync_copy(buf_v,
                                out_hbm.at[pl.ds(r0, wb), pl.ds(cid * dh, dh)])

            return carry

        lax.fori_loop(0, per_wb, wcopy, 0)

    return k(ac, dst)


def kernel(node, edge, edge_index, Wn, bn, We, be, Wa, ba, Wt, bt):
    n, d = node.shape
    src = edge_index[0]
    dst = edge_index[1]
    We_top = We[:d]
    We_bot = We[d:]
    P = _node_precompute(node, Wn, bn.reshape(1, d), We_top)
    G = _gather_rows(P, src)
    ac = _edge_compute(G, edge, We_bot, be.reshape(1, d), Wt, bt.reshape(1, d))
    return _scatter_add_relu(ac, dst, n)


# speedup vs baseline: 3.0596x; 1.2863x over previous
import functools

import numpy as np

import jax
import jax.numpy as jnp
from jax import lax
from jax.experimental import pallas as pl
from jax.experimental.pallas import tpu as pltpu
from jax.experimental.pallas import tpu_sc as plsc

_NC = 2
_NS = 16
_CH = 128
_HI = -65536


def _node_precompute(node, Wn, bn2, We_top_p):
    n, d = node.shape
    blk = 1000

    def body(node_ref, wn_ref, bn_ref, wet_ref, out_ref):
        xb = node_ref[...].astype(jnp.bfloat16)
        wn = wn_ref[...].astype(jnp.bfloat16)
        h = jnp.dot(xb, wn, preferred_element_type=jnp.float32) + bn_ref[...]
        h = jnp.where(h >= 0, h, 0.01 * h)
        wet = wet_ref[...].astype(jnp.bfloat16)
        p = jnp.dot(h.astype(jnp.bfloat16), wet,
                    preferred_element_type=jnp.float32)
        a = p[:, : d // 2].astype(jnp.bfloat16).astype(jnp.float32)
        b = p[:, d // 2:].astype(jnp.bfloat16).astype(jnp.float32)
        ai = jax.lax.bitcast_convert_type(a, jnp.int32)
        bi = jax.lax.bitcast_convert_type(b, jnp.int32)
        hi = jnp.int32(_HI)
        out_ref[...] = jax.lax.shift_right_logical(ai, 16) | (bi & hi)

    return pl.pallas_call(
        body,
        grid=(n // blk,),
        in_specs=[
            pl.BlockSpec((blk, d), lambda i: (i, 0)),
            pl.BlockSpec((d, d), lambda i: (0, 0)),
            pl.BlockSpec((1, d), lambda i: (0, 0)),
            pl.BlockSpec((d, d), lambda i: (0, 0)),
        ],
        out_specs=pl.BlockSpec((blk, d // 2), lambda i: (i, 0)),
        out_shape=jax.ShapeDtypeStruct((n, d // 2), jnp.int32),
    )(node, Wn, bn2, We_top_p)


def _gather_rows(table, idx):
    n, dp = table.shape
    (e,) = idx.shape
    n_chunks = e // _CH
    assert n_chunks * _CH == e
    nw = _NC * _NS
    per_w = (n_chunks + nw - 1) // nw
    per_w += per_w % 2
    mesh = plsc.VectorSubcoreMesh(core_axis_name="c", subcore_axis_name="s")

    @functools.partial(
        pl.kernel,
        out_type=jax.ShapeDtypeStruct((e, dp), jnp.int32),
        mesh=mesh,
        scratch_types=[
            pltpu.VMEM((_CH,), jnp.int32),
            pltpu.VMEM((_CH,), jnp.int32),
            pltpu.VMEM((_CH, dp), jnp.int32),
            pltpu.VMEM((_CH, dp), jnp.int32),
            pltpu.SemaphoreType.DMA,
            pltpu.SemaphoreType.DMA,
            pltpu.SemaphoreType.DMA,
        ],
    )
    def k(table_hbm, idx_hbm, out_hbm, idx0, idx1, rows0, rows1, sg, sw0, sw1):
        wid = lax.axis_index("s") * _NC + lax.axis_index("c")
        idx_b = (idx0, idx1)
        rows_b = (rows0, rows1)
        sw = (sw0, sw1)
        my_n = (n_chunks - wid + nw - 1) // nw

        def outer(i2, carry):
            for b in range(2):
                i = i2 * 2 + b
                c = i * nw + wid

                @pl.when(c < n_chunks)
                def _(b=b, i=i, c=c):
                    @pl.when(i >= 2)
                    def _():
                        pltpu.make_async_copy(
                            rows_b[b], out_hbm.at[pl.ds(0, _CH)], sw[b]).wait()

                    base = c * _CH
                    pltpu.sync_copy(idx_hbm.at[pl.ds(base, _CH)], idx_b[b])
                    pltpu.async_copy(
                        table_hbm.at[idx_b[b]], rows_b[b], sg).wait()
                    pltpu.async_copy(
                        rows_b[b], out_hbm.at[pl.ds(base, _CH)], sw[b])

            return carry

        lax.fori_loop(0, per_w // 2, outer, 0)
        for b in range(2):
            @pl.when((my_n >= 2) | ((my_n == 1) & (b == 0)))
            def _(b=b):
                pltpu.make_async_copy(
                    rows_b[b], out_hbm.at[pl.ds(0, _CH)], sw[b]).wait()

    return k(table, idx)


def _edge_compute(G, edge, We_bot_p, be2_p, Wt_p, bt2):
    e, d = edge.shape
    blk = 1280

    def body(g_ref, e_ref, web_ref, be_ref, wt_ref, bt_ref, out_ref):
        eb = e_ref[...].astype(jnp.bfloat16)
        web = web_ref[...].astype(jnp.bfloat16)
        m = jnp.dot(eb, web, preferred_element_type=jnp.float32)
        g = g_ref[...]
        ge = jax.lax.bitcast_convert_type(g << 16, jnp.float32)
        go = jax.lax.bitcast_convert_type(g & jnp.int32(_HI), jnp.float32)
        m = m + jnp.concatenate([ge, go], axis=1) + be_ref[...]
        m = jnp.where(m >= 0, m, 0.01 * m)
        wt = wt_ref[...].astype(jnp.bfloat16)
        out_ref[...] = jnp.dot(m.astype(jnp.bfloat16), wt,
                               preferred_element_type=jnp.float32) + bt_ref[...]

    return pl.pallas_call(
        body,
        grid=(e // blk,),
        in_specs=[
            pl.BlockSpec((blk, d // 2), lambda i: (i, 0)),
            pl.BlockSpec((blk, d), lambda i: (i, 0)),
            pl.BlockSpec((d, d), lambda i: (0, 0)),
            pl.BlockSpec((1, d), lambda i: (0, 0)),
            pl.BlockSpec((d, d), lambda i: (0, 0)),
            pl.BlockSpec((1, d), lambda i: (0, 0)),
        ],
        out_specs=pl.BlockSpec((blk, d), lambda i: (i, 0)),
        out_shape=jax.ShapeDtypeStruct((e, d), jnp.float32),
    )(G, edge, We_bot_p, be2_p, Wt_p, bt2)


def _scatter_add_relu(ac, dst, n):
    e, d = ac.shape
    dh = d // _NC
    n_chunks = e // _CH
    assert n_chunks * _CH == e
    per_t = (n_chunks + _NS - 1) // _NS
    per_t += per_t % 2
    wb = 80
    n_blk = n // wb
    assert n_blk * wb == n
    per_wb = (n_blk + _NS - 1) // _NS
    mesh = plsc.VectorSubcoreMesh(core_axis_name="c", subcore_axis_name="s")

    @functools.partial(
        pl.kernel,
        out_type=jax.ShapeDtypeStruct((n, d), jnp.float32),
        mesh=mesh,
        scratch_types=[
            pltpu.VMEM((_CH,), jnp.int32),
            pltpu.VMEM((_CH,), jnp.int32),
            pltpu.VMEM((_CH, dh), jnp.float32),
            pltpu.VMEM((_CH, dh), jnp.float32),
            pltpu.VMEM((wb, dh), jnp.float32),
            pltpu.VMEM_SHARED((n, dh), jnp.float32),
            pltpu.SemaphoreType.DMA,
            pltpu.SemaphoreType.DMA,
            pltpu.SemaphoreType.DMA,
        ],
    )
    def k(ac_hbm, dst_hbm, out_hbm, idx0, idx1, rows0, rows1, buf_v, acc_sh,
          sl, sa0, sa1):
        cid = lax.axis_index("c")
        sid = lax.axis_index("s")
        idx_b = (idx0, idx1)
        rows_b = (rows0, rows1)
        sa = (sa0, sa1)
        my_n = (n_chunks - sid + _NS - 1) // _NS

        def zbody(i, carry):
            for j in range(dh // 16):
                buf_v[i, pl.ds(j * 16, 16)] = jnp.zeros((16,), jnp.float32)
            return carry

        lax.fori_loop(0, wb, zbody, 0)

        def zcopy(i, carry):
            g = i * _NS + sid

            @pl.when(g < n_blk)
            def _():
                pltpu.sync_copy(buf_v, acc_sh.at[pl.ds(g * wb, wb)])

            return carry

        lax.fori_loop(0, per_wb, zcopy, 0)
        plsc.subcore_barrier()

        def outer(i2, carry):
            for b in range(2):
                i = i2 * 2 + b
                c = i * _NS + sid

                @pl.when(c < n_chunks)
                def _(b=b, i=i, c=c):
                    @pl.when(i >= 2)
                    def _():
                        pltpu.make_async_copy(
                            rows_b[b], acc_sh.at[idx_b[b]], sa[b]).wait()

                    base = c * _CH
                    pltpu.sync_copy(dst_hbm.at[pl.ds(base, _CH)], idx_b[b])
                    pltpu.async_copy(
                        ac_hbm.at[pl.ds(base, _CH), pl.ds(cid * dh, dh)],
                        rows_b[b], sl).wait()
                    pltpu.async_copy(
                        rows_b[b], acc_sh.at[idx_b[b]], sa[b], add=True)

            return carry

        lax.fori_loop(0, per_t // 2, outer, 0)
        for b in range(2):
            @pl.when((my_n >= 2) | ((my_n == 1) & (b == 0)))
            def _(b=b):
                pltpu.make_async_copy(
                    rows_b[b], acc_sh.at[idx_b[b]], sa[b]).wait()

        plsc.subcore_barrier()

        def wcopy(i, carry):
            g = i * _NS + sid

            @pl.when(g < n_blk)
            def _():
                r0 = g * wb
                pltpu.sync_copy(acc_sh.at[pl.ds(r0, wb)], buf_v)

                def rbody(ii, cc):
                    for j in range(dh // 16):
                        s = pl.ds(j * 16, 16)
                        buf_v[ii, s] = jnp.maximum(buf_v[ii, s], 0.0)
                    return cc

                lax.fori_loop(0, wb, rbody, 0)
                pltpu.sync_copy(buf_v,
                                out_hbm.at[pl.ds(r0, wb), pl.ds(cid * dh, dh)])

            return carry

        lax.fori_loop(0, per_wb, wcopy, 0)

    return k(ac, dst)


def kernel(node, edge, edge_index, Wn, bn, We, be, Wa, ba, Wt, bt):
    n, d = node.shape
    src = edge_index[0]
    dst = edge_index[1]
    perm = np.concatenate([np.arange(0, d, 2), np.arange(1, d, 2)])
    We_top_p = We[:d][:, perm]
    We_bot_p = We[d:][:, perm]
    be_p = be[perm]
    Wt_p = Wt[perm, :]
    P = _node_precompute(node, Wn, bn.reshape(1, d), We_top_p)
    G = _gather_rows(P, src)
    ac = _edge_compute(G, edge, We_bot_p, be_p.reshape(1, d), Wt_p,
                       bt.reshape(1, d))
    return _scatter_add_relu(ac, dst, n)
